# baseline (speedup 1.0000x reference)
import functools

import jax
import jax.numpy as jnp
from jax import lax
from jax.experimental import pallas as pl
from jax.experimental.pallas import tpu as pltpu
from jax.experimental.pallas import tpu_sc as plsc

B = 16384
F = 100
EMB = 16
D_IN = F * EMB
EPS = 1e-5

F_PAD = 104
D_PAD = F_PAD * EMB
N_OUT = B * F_PAD

NC, NS = 2, 16
NW = NC * NS
BROWS_W = B // NW
G = 8
CHUNK = G * F_PAD
NCHUNK = BROWS_W // G


def _sc_gather(table, idx):
    mesh = plsc.VectorSubcoreMesh(core_axis_name="c", subcore_axis_name="s")

    @functools.partial(
        pl.kernel,
        out_type=jax.ShapeDtypeStruct((N_OUT, EMB), jnp.float32),
        mesh=mesh,
        scratch_types=[
            pltpu.VMEM((BROWS_W, F_PAD), jnp.int32),
            pltpu.VMEM((CHUNK, EMB), jnp.float32),
            pltpu.VMEM((CHUNK, EMB), jnp.float32),
            pltpu.SemaphoreType.DMA,
            pltpu.SemaphoreType.DMA,
            pltpu.SemaphoreType.DMA,
            pltpu.SemaphoreType.DMA,
        ],
        compiler_params=pltpu.CompilerParams(use_tc_tiling_on_sc=False),
    )
    def gather_kernel(table_hbm, idx_hbm, out_hbm, idx_v,
                      rows0, rows1, gsem0, gsem1, osem0, osem1):
        wid = lax.axis_index("s") * NC + lax.axis_index("c")
        pltpu.sync_copy(idx_hbm.at[pl.ds(wid * BROWS_W, BROWS_W)], idx_v)

        row_base = wid * BROWS_W * F_PAD
        rows = (rows0, rows1)
        gsem = (gsem0, gsem1)
        osem = (osem0, osem1)

        def fire(c, b):
            for g in range(G):
                pltpu.async_copy(table_hbm.at[idx_v.at[c * G + g]],
                                 rows[b].at[pl.ds(g * F_PAD, F_PAD)], gsem[b])

        def drain_g(b):
            pltpu.make_async_copy(table_hbm.at[pl.ds(0, CHUNK)],
                                  rows[b], gsem[b]).wait()

        def drain_o(b):
            pltpu.make_async_copy(rows[b], out_hbm.at[pl.ds(0, CHUNK)],
                                  osem[b]).wait()

        fire(0, 0)

        @pl.loop(0, NCHUNK // 2)
        def _pair(p):
            for b in range(2):
                c = 2 * p + b

                @pl.when(c >= 1)
                def _():
                    drain_o(1 - b)

                @pl.when(c + 1 < NCHUNK)
                def _():
                    fire(c + 1, 1 - b)

                drain_g(b)
                pltpu.async_copy(
                    rows[b], out_hbm.at[pl.ds(row_base + c * CHUNK, CHUNK)],
                    osem[b])

        drain_o(1)

    return gather_kernel(table, idx)


def _mlp_body(emb, s, t, w1, b1, w2, b2, w3, b3, out):
    a = (emb[...] * s[...] + t[...]).astype(jnp.bfloat16)
    h = jnp.dot(a, w1[...], preferred_element_type=jnp.float32) + b1[...]
    h = jnp.maximum(h, 0.0).astype(jnp.bfloat16)
    h = jnp.dot(h, w2[...], preferred_element_type=jnp.float32) + b2[...]
    h = jnp.maximum(h, 0.0).astype(jnp.bfloat16)
    z = jnp.dot(h, w3[...], preferred_element_type=jnp.float32)
    out[...] = jax.nn.sigmoid(z[:, 0:1] + b3[...])


def _tc_mlp(emb, s, t, w1, b1, w2, b2, w3, b3, bm=512):
    grid = (B // bm,)
    return pl.pallas_call(
        _mlp_body,
        grid=grid,
        in_specs=[
            pl.BlockSpec((bm, D_PAD), lambda i: (i, 0)),
            pl.BlockSpec((1, D_PAD), lambda i: (0, 0)),
            pl.BlockSpec((1, D_PAD), lambda i: (0, 0)),
            pl.BlockSpec((D_PAD, 1024), lambda i: (0, 0)),
            pl.BlockSpec((1, 1024), lambda i: (0, 0)),
            pl.BlockSpec((1024, 512), lambda i: (0, 0)),
            pl.BlockSpec((1, 512), lambda i: (0, 0)),
            pl.BlockSpec((512, 128), lambda i: (0, 0)),
            pl.BlockSpec((1, 1), lambda i: (0, 0)),
        ],
        out_specs=pl.BlockSpec((bm, 1), lambda i: (i, 0)),
        out_shape=jax.ShapeDtypeStruct((B, 1), jnp.float32),
    )(emb, s, t, w1, b1, w2, b2, w3, b3)


def kernel(x, table, rm, rv, gamma, beta, W1, b1, W2, b2, W3, b3):
    idx = jnp.pad(x.astype(jnp.int32), ((0, 0), (0, F_PAD - F)))
    emb = _sc_gather(table, idx).reshape(B, D_PAD)
    inv = lax.rsqrt(rv + EPS)
    s = jnp.pad((gamma * inv), (0, D_PAD - D_IN)).reshape(1, D_PAD)
    t = jnp.pad((beta - rm * gamma * inv), (0, D_PAD - D_IN)).reshape(1, D_PAD)
    w1 = jnp.pad(W1, ((0, D_PAD - D_IN), (0, 0))).astype(jnp.bfloat16)
    w2 = W2.astype(jnp.bfloat16)
    w3 = jnp.pad(W3, ((0, 0), (0, 127))).astype(jnp.bfloat16)
    return _tc_mlp(emb, s, t, w1, b1.reshape(1, -1), w2, b2.reshape(1, -1),
                   w3, b3.reshape(1, 1))


# speedup vs baseline: 1.9541x; 1.9541x over previous
import functools

import jax
import jax.numpy as jnp
from jax import lax
from jax.experimental import pallas as pl
from jax.experimental.pallas import tpu as pltpu
from jax.experimental.pallas import tpu_sc as plsc

B = 16384
F = 100
EMB = 16
D_IN = F * EMB
EPS = 1e-5

NC, NS = 2, 16
NW = NC * NS
IDX_W = 128
GPC = 4
CHUNK = GPC * IDX_W

NSPLIT = 2
BS = B // NSPLIT


def _sc_gather(table, idx2d):
    n_idx_rows = idx2d.shape[0]
    n_rows = n_idx_rows * IDX_W
    irows_w = n_idx_rows // NW
    rows_w = n_rows // NW
    nchunk = rows_w // CHUNK
    assert nchunk % 2 == 0 and nchunk * CHUNK == rows_w

    mesh = plsc.VectorSubcoreMesh(core_axis_name="c", subcore_axis_name="s")

    @functools.partial(
        pl.kernel,
        out_type=jax.ShapeDtypeStruct((n_rows, EMB), jnp.float32),
        mesh=mesh,
        scratch_types=[
            pltpu.VMEM((irows_w, IDX_W), jnp.int32),
            pltpu.VMEM((CHUNK, EMB), jnp.float32),
            pltpu.VMEM((CHUNK, EMB), jnp.float32),
            pltpu.SemaphoreType.DMA,
            pltpu.SemaphoreType.DMA,
            pltpu.SemaphoreType.DMA,
            pltpu.SemaphoreType.DMA,
        ],
        compiler_params=pltpu.CompilerParams(use_tc_tiling_on_sc=False),
    )
    def gather_kernel(table_hbm, idx_hbm, out_hbm, idx_v,
                      rows0, rows1, gsem0, gsem1, osem0, osem1):
        wid = lax.axis_index("s") * NC + lax.axis_index("c")
        pltpu.sync_copy(idx_hbm.at[pl.ds(wid * irows_w, irows_w)], idx_v)
        row_base = wid * rows_w
        rows = (rows0, rows1)
        gsem = (gsem0, gsem1)
        osem = (osem0, osem1)

        def fire(c, b):
            for j in range(GPC):
                pltpu.async_copy(table_hbm.at[idx_v.at[c * GPC + j]],
                                 rows[b].at[pl.ds(j * IDX_W, IDX_W)], gsem[b])

        def drain_g(b):
            pltpu.make_async_copy(table_hbm.at[pl.ds(0, CHUNK)],
                                  rows[b], gsem[b]).wait()

        def drain_o(b):
            pltpu.make_async_copy(rows[b], out_hbm.at[pl.ds(0, CHUNK)],
                                  osem[b]).wait()

        fire(0, 0)

        @pl.loop(0, nchunk // 2)
        def _pair(g):
            for b in range(2):
                c = 2 * g + b

                @pl.when(c >= 1)
                def _():
                    drain_o(1 - b)

                @pl.when(c + 1 < nchunk)
                def _():
                    fire(c + 1, 1 - b)

                drain_g(b)
                pltpu.async_copy(
                    rows[b], out_hbm.at[pl.ds(row_base + c * CHUNK, CHUNK)],
                    osem[b])

        drain_o(1)

    return gather_kernel(table, idx2d)


def _mlp_body(emb, s, t, w1, b1, w2, b2, w3, b3, out):
    a = (emb[...] * s[...] + t[...]).astype(jnp.bfloat16)
    h = jnp.dot(a, w1[...], preferred_element_type=jnp.float32) + b1[...]
    h = jnp.maximum(h, 0.0).astype(jnp.bfloat16)
    h = jnp.dot(h, w2[...], preferred_element_type=jnp.float32) + b2[...]
    h = jnp.maximum(h, 0.0).astype(jnp.bfloat16)
    z = jnp.dot(h, w3[...], preferred_element_type=jnp.float32)
    out[...] = jax.nn.sigmoid(z[:, 0:1] + b3[...])


def _tc_mlp(emb, s, t, w1, b1, w2, b2, w3, b3, bm=512):
    nb = emb.shape[0]
    return pl.pallas_call(
        _mlp_body,
        grid=(nb // bm,),
        in_specs=[
            pl.BlockSpec((bm, D_IN), lambda i: (i, 0)),
            pl.BlockSpec((1, D_IN), lambda i: (0, 0)),
            pl.BlockSpec((1, D_IN), lambda i: (0, 0)),
            pl.BlockSpec((D_IN, 1024), lambda i: (0, 0)),
            pl.BlockSpec((1, 1024), lambda i: (0, 0)),
            pl.BlockSpec((1024, 512), lambda i: (0, 0)),
            pl.BlockSpec((1, 512), lambda i: (0, 0)),
            pl.BlockSpec((512, 128), lambda i: (0, 0)),
            pl.BlockSpec((1, 1), lambda i: (0, 0)),
        ],
        out_specs=pl.BlockSpec((bm, 1), lambda i: (i, 0)),
        out_shape=jax.ShapeDtypeStruct((nb, 1), jnp.float32),
    )(emb, s, t, w1, b1, w2, b2, w3, b3)


def kernel(x, table, rm, rv, gamma, beta, W1, b1, W2, b2, W3, b3):
    idx2d = x.astype(jnp.int32).reshape(B * F // IDX_W, IDX_W)
    inv = lax.rsqrt(rv + EPS)
    s = (gamma * inv).reshape(1, D_IN)
    t = (beta - rm * gamma * inv).reshape(1, D_IN)
    w1 = W1.astype(jnp.bfloat16)
    w2 = W2.astype(jnp.bfloat16)
    w3 = jnp.pad(W3, ((0, 0), (0, 127))).astype(jnp.bfloat16)
    b1r, b2r, b3r = b1.reshape(1, -1), b2.reshape(1, -1), b3.reshape(1, 1)

    irows_half = B * F // IDX_W // NSPLIT
    outs = []
    for k in range(NSPLIT):
        emb = _sc_gather(table, lax.slice_in_dim(idx2d, k * irows_half,
                                                 (k + 1) * irows_half))
        outs.append(_tc_mlp(emb.reshape(BS, D_IN), s, t, w1, b1r, w2, b2r,
                            w3, b3r))
    return jnp.concatenate(outs, axis=0)


# speedup vs baseline: 1.9673x; 1.0067x over previous
import functools

import jax
import jax.numpy as jnp
from jax import lax
from jax.experimental import pallas as pl
from jax.experimental.pallas import tpu as pltpu
from jax.experimental.pallas import tpu_sc as plsc

B = 16384
F = 100
EMB = 16
D_IN = F * EMB
EPS = 1e-5

NC, NS = 2, 16
NW = NC * NS
IDX_W = 128
GPC = 4
CHUNK = GPC * IDX_W

NSPLIT = 2
BS = B // NSPLIT


def _sc_gather(table, idx2d):
    n_idx_rows = idx2d.shape[0]
    n_rows = n_idx_rows * IDX_W
    irows_w = n_idx_rows // NW
    rows_w = n_rows // NW
    nchunk = rows_w // CHUNK
    assert nchunk % 2 == 0 and nchunk * CHUNK == rows_w

    mesh = plsc.VectorSubcoreMesh(core_axis_name="c", subcore_axis_name="s")

    @functools.partial(
        pl.kernel,
        out_type=jax.ShapeDtypeStruct((n_rows, EMB), jnp.float32),
        mesh=mesh,
        scratch_types=[
            pltpu.VMEM((irows_w, IDX_W), jnp.int32),
            pltpu.VMEM((CHUNK, EMB), jnp.float32),
            pltpu.VMEM((CHUNK, EMB), jnp.float32),
            pltpu.SemaphoreType.DMA,
            pltpu.SemaphoreType.DMA,
            pltpu.SemaphoreType.DMA,
            pltpu.SemaphoreType.DMA,
        ],
        compiler_params=pltpu.CompilerParams(use_tc_tiling_on_sc=False),
    )
    def gather_kernel(table_hbm, idx_hbm, out_hbm, idx_v,
                      rows0, rows1, gsem0, gsem1, osem0, osem1):
        wid = lax.axis_index("s") * NC + lax.axis_index("c")
        pltpu.sync_copy(idx_hbm.at[pl.ds(wid * irows_w, irows_w)], idx_v)
        row_base = wid * rows_w
        rows = (rows0, rows1)
        gsem = (gsem0, gsem1)
        osem = (osem0, osem1)

        def fire(c, b):
            for j in range(GPC):
                pltpu.async_copy(table_hbm.at[idx_v.at[c * GPC + j]],
                                 rows[b].at[pl.ds(j * IDX_W, IDX_W)], gsem[b])

        def drain_g(b):
            pltpu.make_async_copy(table_hbm.at[pl.ds(0, CHUNK)],
                                  rows[b], gsem[b]).wait()

        def drain_o(b):
            pltpu.make_async_copy(rows[b], out_hbm.at[pl.ds(0, CHUNK)],
                                  osem[b]).wait()

        fire(0, 0)

        @pl.loop(0, nchunk // 2)
        def _pair(g):
            for b in range(2):
                c = 2 * g + b

                @pl.when(c >= 1)
                def _():
                    drain_o(1 - b)

                @pl.when(c + 1 < nchunk)
                def _():
                    fire(c + 1, 1 - b)

                drain_g(b)
                pltpu.async_copy(
                    rows[b], out_hbm.at[pl.ds(row_base + c * CHUNK, CHUNK)],
                    osem[b])

        drain_o(1)

    return gather_kernel(table, idx2d)


def _mlp_body(emb, s, t, w1, b1, w2, b2, w3, b3, out):
    a = (emb[...] * s[...] + t[...]).astype(jnp.bfloat16)
    h = jnp.dot(a, w1[...], preferred_element_type=jnp.float32) + b1[...]
    h = jnp.maximum(h, 0.0).astype(jnp.bfloat16)
    h = jnp.dot(h, w2[...], preferred_element_type=jnp.float32) + b2[...]
    h = jnp.maximum(h, 0.0).astype(jnp.bfloat16)
    z = jnp.dot(h, w3[...], preferred_element_type=jnp.float32)
    out[...] = jax.nn.sigmoid(z[:, 0:1] + b3[...])


def _tc_mlp(emb, s, t, w1, b1, w2, b2, w3, b3, bm=1024):
    nb = emb.shape[0]
    return pl.pallas_call(
        _mlp_body,
        grid=(nb // bm,),
        in_specs=[
            pl.BlockSpec((bm, D_IN), lambda i: (i, 0)),
            pl.BlockSpec((1, D_IN), lambda i: (0, 0)),
            pl.BlockSpec((1, D_IN), lambda i: (0, 0)),
            pl.BlockSpec((D_IN, 1024), lambda i: (0, 0)),
            pl.BlockSpec((1, 1024), lambda i: (0, 0)),
            pl.BlockSpec((1024, 512), lambda i: (0, 0)),
            pl.BlockSpec((1, 512), lambda i: (0, 0)),
            pl.BlockSpec((512, 128), lambda i: (0, 0)),
            pl.BlockSpec((1, 1), lambda i: (0, 0)),
        ],
        out_specs=pl.BlockSpec((bm, 1), lambda i: (i, 0)),
        out_shape=jax.ShapeDtypeStruct((nb, 1), jnp.float32),
    )(emb, s, t, w1, b1, w2, b2, w3, b3)


def kernel(x, table, rm, rv, gamma, beta, W1, b1, W2, b2, W3, b3):
    idx2d = x.astype(jnp.int32).reshape(B * F // IDX_W, IDX_W)
    inv = lax.rsqrt(rv + EPS)
    s = (gamma * inv).reshape(1, D_IN)
    t = (beta - rm * gamma * inv).reshape(1, D_IN)
    w1 = W1.astype(jnp.bfloat16)
    w2 = W2.astype(jnp.bfloat16)
    w3 = jnp.pad(W3, ((0, 0), (0, 127))).astype(jnp.bfloat16)
    b1r, b2r, b3r = b1.reshape(1, -1), b2.reshape(1, -1), b3.reshape(1, 1)

    irows_half = B * F // IDX_W // NSPLIT
    outs = []
    for k in range(NSPLIT):
        emb = _sc_gather(table, lax.slice_in_dim(idx2d, k * irows_half,
                                                 (k + 1) * irows_half))
        outs.append(_tc_mlp(emb.reshape(BS, D_IN), s, t, w1, b1r, w2, b2r,
                            w3, b3r))
    return jnp.concatenate(outs, axis=0)


# speedup vs baseline: 1.9975x; 1.0154x over previous
import functools

import jax
import jax.numpy as jnp
from jax import lax
from jax.experimental import pallas as pl
from jax.experimental.pallas import tpu as pltpu
from jax.experimental.pallas import tpu_sc as plsc

B = 16384
F = 100
EMB = 16
D_IN = F * EMB
EPS = 1e-5

NC, NS = 2, 16
NW = NC * NS
IDX_W = 128
GPC = 5
CHUNK = GPC * IDX_W

NSPLIT = 4
BS = B // NSPLIT


def _sc_gather(table, idx2d):
    n_idx_rows = idx2d.shape[0]
    n_rows = n_idx_rows * IDX_W
    irows_w = n_idx_rows // NW
    rows_w = n_rows // NW
    nchunk = rows_w // CHUNK
    assert nchunk % 2 == 0 and nchunk * CHUNK == rows_w

    mesh = plsc.VectorSubcoreMesh(core_axis_name="c", subcore_axis_name="s")

    @functools.partial(
        pl.kernel,
        out_type=jax.ShapeDtypeStruct((n_rows, EMB), jnp.float32),
        mesh=mesh,
        scratch_types=[
            pltpu.VMEM((irows_w, IDX_W), jnp.int32),
            pltpu.VMEM((CHUNK, EMB), jnp.float32),
            pltpu.VMEM((CHUNK, EMB), jnp.float32),
            pltpu.SemaphoreType.DMA,
            pltpu.SemaphoreType.DMA,
            pltpu.SemaphoreType.DMA,
            pltpu.SemaphoreType.DMA,
        ],
        compiler_params=pltpu.CompilerParams(use_tc_tiling_on_sc=False),
    )
    def gather_kernel(table_hbm, idx_hbm, out_hbm, idx_v,
                      rows0, rows1, gsem0, gsem1, osem0, osem1):
        wid = lax.axis_index("s") * NC + lax.axis_index("c")
        pltpu.sync_copy(idx_hbm.at[pl.ds(wid * irows_w, irows_w)], idx_v)
        row_base = wid * rows_w
        rows = (rows0, rows1)
        gsem = (gsem0, gsem1)
        osem = (osem0, osem1)

        def fire(c, b):
            for j in range(GPC):
                pltpu.async_copy(table_hbm.at[idx_v.at[c * GPC + j]],
                                 rows[b].at[pl.ds(j * IDX_W, IDX_W)], gsem[b])

        def drain_g(b):
            pltpu.make_async_copy(table_hbm.at[pl.ds(0, CHUNK)],
                                  rows[b], gsem[b]).wait()

        def drain_o(b):
            pltpu.make_async_copy(rows[b], out_hbm.at[pl.ds(0, CHUNK)],
                                  osem[b]).wait()

        fire(0, 0)

        @pl.loop(0, nchunk // 2)
        def _pair(g):
            for b in range(2):
                c = 2 * g + b

                @pl.when(c >= 1)
                def _():
                    drain_o(1 - b)

                @pl.when(c + 1 < nchunk)
                def _():
                    fire(c + 1, 1 - b)

                drain_g(b)
                pltpu.async_copy(
                    rows[b], out_hbm.at[pl.ds(row_base + c * CHUNK, CHUNK)],
                    osem[b])

        drain_o(1)

    return gather_kernel(table, idx2d)


def _mlp_body(emb, s, t, w1, b1, w2, b2, w3, b3, out):
    a = (emb[...] * s[...] + t[...]).astype(jnp.bfloat16)
    h = jnp.dot(a, w1[...], preferred_element_type=jnp.float32) + b1[...]
    h = jnp.maximum(h, 0.0).astype(jnp.bfloat16)
    h = jnp.dot(h, w2[...], preferred_element_type=jnp.float32) + b2[...]
    h = jnp.maximum(h, 0.0).astype(jnp.bfloat16)
    z = jnp.dot(h, w3[...], preferred_element_type=jnp.float32)
    out[...] = jax.nn.sigmoid(z[:, 0:1] + b3[...])


def _tc_mlp(emb, s, t, w1, b1, w2, b2, w3, b3, bm=1024):
    nb = emb.shape[0]
    return pl.pallas_call(
        _mlp_body,
        grid=(nb // bm,),
        in_specs=[
            pl.BlockSpec((bm, D_IN), lambda i: (i, 0)),
            pl.BlockSpec((1, D_IN), lambda i: (0, 0)),
            pl.BlockSpec((1, D_IN), lambda i: (0, 0)),
            pl.BlockSpec((D_IN, 1024), lambda i: (0, 0)),
            pl.BlockSpec((1, 1024), lambda i: (0, 0)),
            pl.BlockSpec((1024, 512), lambda i: (0, 0)),
            pl.BlockSpec((1, 512), lambda i: (0, 0)),
            pl.BlockSpec((512, 128), lambda i: (0, 0)),
            pl.BlockSpec((1, 1), lambda i: (0, 0)),
        ],
        out_specs=pl.BlockSpec((bm, 1), lambda i: (i, 0)),
        out_shape=jax.ShapeDtypeStruct((nb, 1), jnp.float32),
    )(emb, s, t, w1, b1, w2, b2, w3, b3)


def kernel(x, table, rm, rv, gamma, beta, W1, b1, W2, b2, W3, b3):
    idx2d = x.astype(jnp.int32).reshape(B * F // IDX_W, IDX_W)
    inv = lax.rsqrt(rv + EPS)
    s = (gamma * inv).reshape(1, D_IN)
    t = (beta - rm * gamma * inv).reshape(1, D_IN)
    w1 = W1.astype(jnp.bfloat16)
    w2 = W2.astype(jnp.bfloat16)
    w3 = jnp.pad(W3, ((0, 0), (0, 127))).astype(jnp.bfloat16)
    b1r, b2r, b3r = b1.reshape(1, -1), b2.reshape(1, -1), b3.reshape(1, 1)

    irows_half = B * F // IDX_W // NSPLIT
    outs = []
    for k in range(NSPLIT):
        emb = _sc_gather(table, lax.slice_in_dim(idx2d, k * irows_half,
                                                 (k + 1) * irows_half))
        outs.append(_tc_mlp(emb.reshape(BS, D_IN), s, t, w1, b1r, w2, b2r,
                            w3, b3r))
    return jnp.concatenate(outs, axis=0)
